# baseline (speedup 1.0000x reference)
import functools

import jax
import jax.numpy as jnp
from jax import lax
from jax.experimental import pallas as pl
from jax.experimental.pallas import tpu as pltpu
from jax.experimental.pallas import tpu_sc as plsc

N_NODES = 10000
D = 128
N_PAD = 10240
NW = 32
CHUNK = 128
K0 = 160
K1 = 0
EDGES_PER_W = 10240
E_PAD = 16 * CHUNK * (K0 + K1)
DUMMY_DST = N_PAD - 1


def _seg_sum_body(h_hbm, src_hbm, dst_hbm, out_hbm, acc,
                  rows0, rows1, is0, is1, is2, is3, id0, id1, id2, id3,
                  gsem0, gsem1, ssem0, ssem1, isem0, isem1, isem2, isem3):
    c = lax.axis_index("c")
    s = lax.axis_index("s")
    nch = jnp.where(c == 0, K0, K1)
    cbase = jnp.where(c == 0, s * K0, 16 * K0 + s * K1)

    rows = [rows0, rows1]
    isb = [is0, is1, is2, is3]
    idb = [id0, id1, id2, id3]
    gsem = [gsem0, gsem1]
    ssem = [ssem0, ssem1]
    isem = [isem0, isem1, isem2, isem3]

    zero16 = jnp.zeros((16,), jnp.float32)

    def fill(i, _):
        for j in range(8):
            rows0[i, pl.ds(j * 16, 16)] = zero16
        return 0

    lax.fori_loop(0, CHUNK, fill, 0)

    def zslice(i, _):
        pltpu.sync_copy(rows0, acc.at[pl.ds(s * 640 + i * CHUNK, CHUNK)])
        return 0

    lax.fori_loop(0, 5, zslice, 0)
    plsc.subcore_barrier()

    def idx_load(k, j):
        off = (cbase + k) * CHUNK
        pltpu.async_copy(src_hbm.at[pl.ds(off, CHUNK)], isb[j], isem[j])
        pltpu.async_copy(dst_hbm.at[pl.ds(off, CHUNK)], idb[j], isem[j])

    def idx_wait(j):
        pltpu.make_async_copy(src_hbm.at[pl.ds(0, CHUNK)], isb[j], isem[j]).wait()
        pltpu.make_async_copy(dst_hbm.at[pl.ds(0, CHUNK)], idb[j], isem[j]).wait()

    def gather(j, br):
        pltpu.async_copy(h_hbm.at[isb[j]], rows[br], gsem[br])

    def gather_wait(j, br):
        pltpu.make_async_copy(h_hbm.at[isb[j]], rows[br], gsem[br]).wait()

    def scatter(j, br):
        pltpu.async_copy(rows[br], acc.at[idb[j]], ssem[br], add=True)

    def scatter_wait(j, br):
        pltpu.make_async_copy(rows[br], acc.at[idb[j]], ssem[br]).wait()

    @pl.when(nch > 0)
    def _():
        idx_load(0, 0)
        idx_load(1, 1)
        idx_load(2, 2)
        idx_wait(0)
        gather(0, 0)

    def outer(t, _):
        for j4 in range(4):
            g = t * 4 + j4
            br = j4 % 2

            @pl.when(g > 0)
            def _():
                scatter_wait((j4 - 1) % 4, (j4 + 1) % 2)

            @pl.when(g + 3 < nch)
            def _():
                idx_load(g + 3, (j4 + 3) % 4)

            gather_wait(j4, br)
            scatter(j4, br)

            @pl.when(g + 1 < nch)
            def _():
                idx_wait((j4 + 1) % 4)
                gather((j4 + 1) % 4, (j4 + 1) % 2)
        return 0

    lax.fori_loop(0, nch // 4, outer, 0)
    @pl.when(nch > 0)
    def _():
        scatter_wait(3, 1)

    plsc.subcore_barrier()

    def wslice(i, _):
        base = s * 640 + i * CHUNK
        pltpu.sync_copy(acc.at[pl.ds(base, CHUNK)], rows0)
        pltpu.sync_copy(rows0, out_hbm.at[c].at[pl.ds(base, CHUNK)])
        return 0

    lax.fori_loop(0, 5, wslice, 0)


def _deg_body(dst_hbm, deg_hbm, dacc, buf, idst, sem):
    del sem
    c = lax.axis_index("c")
    s = lax.axis_index("s")
    wid = c * 16 + s

    zero16 = jnp.zeros((16,), jnp.float32)
    one16 = jnp.ones((16,), jnp.float32)

    def fill_zero(i, _):
        for j in range(8):
            buf[i, pl.ds(j * 16, 16)] = zero16
        return 0

    def fill_one(i, _):
        for j in range(8):
            buf[i, pl.ds(j * 16, 16)] = one16
        return 0

    lax.fori_loop(0, CHUNK, fill_zero, 0)

    def zslice(i, _):
        pltpu.sync_copy(buf, dacc.at[pl.ds(s * 640 + i * CHUNK, CHUNK)])
        return 0

    lax.fori_loop(0, 5, zslice, 0)
    lax.fori_loop(0, CHUNK, fill_one, 0)
    plsc.subcore_barrier()

    ebase = wid * EDGES_PER_W

    def chunk(g, _):
        off = ebase + g * CHUNK
        pltpu.sync_copy(dst_hbm.at[pl.ds(off, CHUNK)], idst)
        pltpu.sync_copy(buf, dacc.at[idst], add=True)
        return 0

    lax.fori_loop(0, EDGES_PER_W // CHUNK, chunk, 0)
    plsc.subcore_barrier()

    def wslice(i, _):
        base = s * 640 + i * CHUNK
        pltpu.sync_copy(dacc.at[pl.ds(base, CHUNK)], buf)
        pltpu.sync_copy(buf, deg_hbm.at[c].at[pl.ds(base, CHUNK)])
        return 0

    lax.fori_loop(0, 5, wslice, 0)


@functools.lru_cache(maxsize=None)
def _make_seg_sum():
    mesh = plsc.VectorSubcoreMesh(core_axis_name="c", subcore_axis_name="s")
    return pl.kernel(
        _seg_sum_body,
        out_type=(jax.ShapeDtypeStruct((2, N_PAD, D), jnp.float32),),
        mesh=mesh,
        scratch_types=(
            [pltpu.VMEM_SHARED((N_PAD, D), jnp.float32)]
            + [pltpu.VMEM((CHUNK, D), jnp.float32)] * 2
            + [pltpu.VMEM((CHUNK,), jnp.int32)] * 8
            + [pltpu.SemaphoreType.DMA] * 8
        ),
    )


@functools.lru_cache(maxsize=None)
def _make_deg():
    mesh = plsc.VectorSubcoreMesh(core_axis_name="c", subcore_axis_name="s")
    return pl.kernel(
        _deg_body,
        out_type=(jax.ShapeDtypeStruct((2, N_PAD, D), jnp.float32),),
        mesh=mesh,
        scratch_types=[
            pltpu.VMEM_SHARED((N_PAD, D), jnp.float32),
            pltpu.VMEM((CHUNK, D), jnp.float32),
            pltpu.VMEM((CHUNK,), jnp.int32),
            pltpu.SemaphoreType.DMA,
        ],
    )


def _layer_body(s_ref, h_ref, d_ref, wm_ref, bm_ref, wua_ref, wub_ref, bu_ref, o_ref):
    h = h_ref[...]
    t = s_ref[0] + s_ref[1] + h
    deg = d_ref[0, :, 0:1] + d_ref[1, :, 0:1] + 1.0
    aggr = jax.lax.dot_general(t, wm_ref[...], (((1,), (0,)), ((), ())),
                               preferred_element_type=jnp.float32)
    aggr = aggr + deg * bm_ref[...]
    u = jax.lax.dot_general(aggr, wua_ref[...], (((1,), (0,)), ((), ())),
                            preferred_element_type=jnp.float32)
    u = u + jax.lax.dot_general(h, wub_ref[...], (((1,), (0,)), ((), ())),
                                preferred_element_type=jnp.float32)
    o_ref[...] = jnp.maximum(u + bu_ref[...], 0.0)


_BLK = 256


def _tc_layer(s_parts, h, deg, Wm, bm, Wu, bu):
    grid = (N_PAD // _BLK,)
    return pl.pallas_call(
        _layer_body,
        grid=grid,
        in_specs=[
            pl.BlockSpec((2, _BLK, D), lambda i: (0, i, 0)),
            pl.BlockSpec((_BLK, D), lambda i: (i, 0)),
            pl.BlockSpec((2, _BLK, D), lambda i: (0, i, 0)),
            pl.BlockSpec((D, D), lambda i: (0, 0)),
            pl.BlockSpec((1, D), lambda i: (0, 0)),
            pl.BlockSpec((D, D), lambda i: (0, 0)),
            pl.BlockSpec((D, D), lambda i: (0, 0)),
            pl.BlockSpec((1, D), lambda i: (0, 0)),
        ],
        out_specs=pl.BlockSpec((_BLK, D), lambda i: (i, 0)),
        out_shape=jax.ShapeDtypeStruct((N_PAD, D), jnp.float32),
    )(s_parts, h, deg, Wm, bm.reshape(1, D), Wu[:D], Wu[D:], bu.reshape(1, D))


def kernel(x, edge_index, Wm0, bm0, Wu0, bu0, Wm1, bm1, Wu1, bu1, Wm2, bm2, Wu2, bu2):
    src = edge_index[0].astype(jnp.int32)
    dst = edge_index[1].astype(jnp.int32)
    n_extra = E_PAD - src.shape[0]
    src = jnp.concatenate([src, jnp.zeros((n_extra,), jnp.int32)])
    dst = jnp.concatenate([dst, jnp.full((n_extra,), DUMMY_DST, jnp.int32)])

    h = jnp.concatenate([x, jnp.zeros((N_PAD - N_NODES, D), x.dtype)])

    (deg,) = _make_deg()(dst)
    (s_parts,) = _make_seg_sum()(h, src, dst)
    h = _tc_layer(s_parts, h, deg, Wm0, bm0, Wu0, bu0)
    (s_parts,) = _make_seg_sum()(h, src, dst)
    h = _tc_layer(s_parts, h, deg, Wm1, bm1, Wu1, bu1)
    (s_parts,) = _make_seg_sum()(h, src, dst)
    h = _tc_layer(s_parts, h, deg, Wm2, bm2, Wu2, bu2)
    return h[:N_NODES]


# speedup vs baseline: 1.9700x; 1.9700x over previous
import functools

import jax
import jax.numpy as jnp
from jax import lax
from jax.experimental import pallas as pl
from jax.experimental.pallas import tpu as pltpu
from jax.experimental.pallas import tpu_sc as plsc

N_NODES = 10000
D = 128
N_PAD = 10240
NW = 32
CHUNK = 120
K0 = 156
K1 = 12
EDGES_PER_W = 16 * CHUNK * (K0 + K1) // 32
E_PAD = 16 * CHUNK * (K0 + K1)
DUMMY_DST = N_PAD - 1


def _seg_sum_body(h_hbm, src_hbm, dst_hbm, out_hbm, acc,
                  rows0, rows1, rows2,
                  is0, is1, is2, is3, is4, is5, id0, id1, id2, id3, id4, id5,
                  gsem0, gsem1, gsem2, ssem0, ssem1, ssem2,
                  isem0, isem1, isem2, isem3, isem4, isem5):
    c = lax.axis_index("c")
    s = lax.axis_index("s")
    nch = jnp.where(c == 0, K0, K1)
    cbase = jnp.where(c == 0, s * K0, 16 * K0 + s * K1)

    rows = [rows0, rows1, rows2]
    isb = [is0, is1, is2, is3, is4, is5]
    idb = [id0, id1, id2, id3, id4, id5]
    gsem = [gsem0, gsem1, gsem2]
    ssem = [ssem0, ssem1, ssem2]
    isem = [isem0, isem1, isem2, isem3, isem4, isem5]

    zero16 = jnp.zeros((16,), jnp.float32)

    def fill(i, _):
        for j in range(8):
            rows0[i, pl.ds(j * 16, 16)] = zero16
        return 0

    lax.fori_loop(0, 80, fill, 0)
    zsrc = rows0.at[pl.ds(0, 80)]

    def zslice(i, _):
        pltpu.sync_copy(zsrc, acc.at[pl.ds(s * 640 + i * 80, 80)])
        return 0

    lax.fori_loop(0, 8, zslice, 0)
    plsc.subcore_barrier()

    def idx_load(k, j):
        off = (cbase + k) * CHUNK
        pltpu.async_copy(src_hbm.at[pl.ds(off, CHUNK)], isb[j], isem[j])
        pltpu.async_copy(dst_hbm.at[pl.ds(off, CHUNK)], idb[j], isem[j])

    def idx_wait(j):
        pltpu.make_async_copy(src_hbm.at[pl.ds(0, CHUNK)], isb[j], isem[j]).wait()
        pltpu.make_async_copy(dst_hbm.at[pl.ds(0, CHUNK)], idb[j], isem[j]).wait()

    def gather(j, br):
        pltpu.async_copy(h_hbm.at[isb[j]], rows[br], gsem[br])

    def gather_wait(j, br):
        pltpu.make_async_copy(h_hbm.at[isb[j]], rows[br], gsem[br]).wait()

    def scatter(j, br):
        pltpu.async_copy(rows[br], acc.at[idb[j]], ssem[br], add=True)

    def scatter_wait(j, br):
        pltpu.make_async_copy(rows[br], acc.at[idb[j]], ssem[br]).wait()

    idx_load(0, 0)
    idx_load(1, 1)
    idx_load(2, 2)
    idx_wait(0)
    gather(0, 0)

    def outer(t, _):
        for j in range(12):
            g = t * 12 + j
            br = j % 3
            bi = j % 6

            @pl.when(g >= 2)
            def _():
                scatter_wait((j - 2) % 6, (j - 2) % 3)

            @pl.when(g + 3 < nch)
            def _():
                idx_load(g + 3, (j + 3) % 6)

            gather_wait(bi, br)
            scatter(bi, br)

            @pl.when(g + 1 < nch)
            def _():
                idx_wait((j + 1) % 6)
                gather((j + 1) % 6, (j + 1) % 3)
        return 0

    lax.fori_loop(0, nch // 12, outer, 0)
    scatter_wait(4, 1)
    scatter_wait(5, 2)
    plsc.subcore_barrier()

    def wslice(i, _):
        base = s * 640 + i * 80
        pltpu.sync_copy(acc.at[pl.ds(base, 80)], zsrc)
        pltpu.sync_copy(zsrc, out_hbm.at[c].at[pl.ds(base, 80)])
        return 0

    lax.fori_loop(0, 8, wslice, 0)


def _deg_body(dst_hbm, deg_hbm, dacc, buf, idst, sem):
    del sem
    c = lax.axis_index("c")
    s = lax.axis_index("s")
    wid = c * 16 + s

    zero16 = jnp.zeros((16,), jnp.float32)
    one16 = jnp.ones((16,), jnp.float32)

    def fill_zero(i, _):
        for j in range(8):
            buf[i, pl.ds(j * 16, 16)] = zero16
        return 0

    def fill_one(i, _):
        for j in range(8):
            buf[i, pl.ds(j * 16, 16)] = one16
        return 0

    lax.fori_loop(0, 80, fill_zero, 0)
    zsrc = buf.at[pl.ds(0, 80)]

    def zslice(i, _):
        pltpu.sync_copy(zsrc, dacc.at[pl.ds(s * 640 + i * 80, 80)])
        return 0

    lax.fori_loop(0, 8, zslice, 0)
    lax.fori_loop(0, CHUNK, fill_one, 0)
    plsc.subcore_barrier()

    ebase = wid * EDGES_PER_W

    def chunk(g, _):
        off = ebase + g * CHUNK
        pltpu.sync_copy(dst_hbm.at[pl.ds(off, CHUNK)], idst)
        pltpu.sync_copy(buf, dacc.at[idst], add=True)
        return 0

    lax.fori_loop(0, EDGES_PER_W // CHUNK, chunk, 0)
    plsc.subcore_barrier()

    def wslice(i, _):
        base = s * 640 + i * 80
        pltpu.sync_copy(dacc.at[pl.ds(base, 80)], zsrc)
        pltpu.sync_copy(zsrc, deg_hbm.at[c].at[pl.ds(base, 80)])
        return 0

    lax.fori_loop(0, 8, wslice, 0)


@functools.lru_cache(maxsize=None)
def _make_seg_sum():
    mesh = plsc.VectorSubcoreMesh(core_axis_name="c", subcore_axis_name="s")
    return pl.kernel(
        _seg_sum_body,
        out_type=(jax.ShapeDtypeStruct((2, N_PAD, D), jnp.float32),),
        mesh=mesh,
        scratch_types=(
            [pltpu.VMEM_SHARED((N_PAD, D), jnp.float32)]
            + [pltpu.VMEM((CHUNK, D), jnp.float32)] * 3
            + [pltpu.VMEM((CHUNK,), jnp.int32)] * 12
            + [pltpu.SemaphoreType.DMA] * 12
        ),
    )


@functools.lru_cache(maxsize=None)
def _make_deg():
    mesh = plsc.VectorSubcoreMesh(core_axis_name="c", subcore_axis_name="s")
    return pl.kernel(
        _deg_body,
        out_type=(jax.ShapeDtypeStruct((2, N_PAD, D), jnp.float32),),
        mesh=mesh,
        scratch_types=[
            pltpu.VMEM_SHARED((N_PAD, D), jnp.float32),
            pltpu.VMEM((CHUNK, D), jnp.float32),
            pltpu.VMEM((CHUNK,), jnp.int32),
            pltpu.SemaphoreType.DMA,
        ],
    )


def _layer_body(s_ref, h_ref, d_ref, wm_ref, bm_ref, wua_ref, wub_ref, bu_ref, o_ref):
    h = h_ref[...]
    t = s_ref[0] + s_ref[1] + h
    deg = d_ref[0, :, 0:1] + d_ref[1, :, 0:1] + 1.0
    aggr = jax.lax.dot_general(t, wm_ref[...], (((1,), (0,)), ((), ())),
                               preferred_element_type=jnp.float32)
    aggr = aggr + deg * bm_ref[...]
    u = jax.lax.dot_general(aggr, wua_ref[...], (((1,), (0,)), ((), ())),
                            preferred_element_type=jnp.float32)
    u = u + jax.lax.dot_general(h, wub_ref[...], (((1,), (0,)), ((), ())),
                                preferred_element_type=jnp.float32)
    o_ref[...] = jnp.maximum(u + bu_ref[...], 0.0)


_BLK = 256


def _tc_layer(s_parts, h, deg, Wm, bm, Wu, bu):
    grid = (N_PAD // _BLK,)
    return pl.pallas_call(
        _layer_body,
        grid=grid,
        in_specs=[
            pl.BlockSpec((2, _BLK, D), lambda i: (0, i, 0)),
            pl.BlockSpec((_BLK, D), lambda i: (i, 0)),
            pl.BlockSpec((2, _BLK, D), lambda i: (0, i, 0)),
            pl.BlockSpec((D, D), lambda i: (0, 0)),
            pl.BlockSpec((1, D), lambda i: (0, 0)),
            pl.BlockSpec((D, D), lambda i: (0, 0)),
            pl.BlockSpec((D, D), lambda i: (0, 0)),
            pl.BlockSpec((1, D), lambda i: (0, 0)),
        ],
        out_specs=pl.BlockSpec((_BLK, D), lambda i: (i, 0)),
        out_shape=jax.ShapeDtypeStruct((N_PAD, D), jnp.float32),
    )(s_parts, h, deg, Wm, bm.reshape(1, D), Wu[:D], Wu[D:], bu.reshape(1, D))


def kernel(x, edge_index, Wm0, bm0, Wu0, bu0, Wm1, bm1, Wu1, bu1, Wm2, bm2, Wu2, bu2):
    src = edge_index[0].astype(jnp.int32)
    dst = edge_index[1].astype(jnp.int32)
    n_extra = E_PAD - src.shape[0]
    src = jnp.concatenate([src, jnp.zeros((n_extra,), jnp.int32)])
    dst = jnp.concatenate([dst, jnp.full((n_extra,), DUMMY_DST, jnp.int32)])

    h = jnp.concatenate([x, jnp.zeros((N_PAD - N_NODES, D), x.dtype)])

    (deg,) = _make_deg()(dst)
    (s_parts,) = _make_seg_sum()(h, src, dst)
    h = _tc_layer(s_parts, h, deg, Wm0, bm0, Wu0, bu0)
    (s_parts,) = _make_seg_sum()(h, src, dst)
    h = _tc_layer(s_parts, h, deg, Wm1, bm1, Wu1, bu1)
    (s_parts,) = _make_seg_sum()(h, src, dst)
    h = _tc_layer(s_parts, h, deg, Wm2, bm2, Wu2, bu2)
    return h[:N_NODES]


# speedup vs baseline: 2.0405x; 1.0358x over previous
import functools

import jax
import jax.numpy as jnp
from jax import lax
from jax.experimental import pallas as pl
from jax.experimental.pallas import tpu as pltpu
from jax.experimental.pallas import tpu_sc as plsc

N_NODES = 10000
D = 128
N_PAD = 10240
NW = 32
CHUNK = 120
K0 = 144
K1 = 24
EDGES_PER_W = 16 * CHUNK * (K0 + K1) // 32
E_PAD = 16 * CHUNK * (K0 + K1)
DUMMY_DST = N_PAD - 1


def _seg_sum_body(h_hbm, src_hbm, dst_hbm, out_hbm, acc,
                  rows0, rows1, rows2,
                  is0, is1, is2, is3, is4, is5, id0, id1, id2, id3, id4, id5,
                  gsem0, gsem1, gsem2, ssem0, ssem1, ssem2,
                  isem0, isem1, isem2, isem3, isem4, isem5):
    c = lax.axis_index("c")
    s = lax.axis_index("s")
    nch = jnp.where(c == 0, K0, K1)
    cbase = jnp.where(c == 0, s * K0, 16 * K0 + s * K1)

    rows = [rows0, rows1, rows2]
    isb = [is0, is1, is2, is3, is4, is5]
    idb = [id0, id1, id2, id3, id4, id5]
    gsem = [gsem0, gsem1, gsem2]
    ssem = [ssem0, ssem1, ssem2]
    isem = [isem0, isem1, isem2, isem3, isem4, isem5]

    zero16 = jnp.zeros((16,), jnp.float32)

    def fill(i, _):
        for j in range(8):
            rows0[i, pl.ds(j * 16, 16)] = zero16
        return 0

    lax.fori_loop(0, 80, fill, 0)
    zsrc = rows0.at[pl.ds(0, 80)]

    def zslice(i, _):
        pltpu.sync_copy(zsrc, acc.at[pl.ds(s * 640 + i * 80, 80)])
        return 0

    lax.fori_loop(0, 8, zslice, 0)
    plsc.subcore_barrier()

    def idx_load(k, j):
        off = (cbase + k) * CHUNK
        pltpu.async_copy(src_hbm.at[pl.ds(off, CHUNK)], isb[j], isem[j])
        pltpu.async_copy(dst_hbm.at[pl.ds(off, CHUNK)], idb[j], isem[j])

    def idx_wait(j):
        pltpu.make_async_copy(src_hbm.at[pl.ds(0, CHUNK)], isb[j], isem[j]).wait()
        pltpu.make_async_copy(dst_hbm.at[pl.ds(0, CHUNK)], idb[j], isem[j]).wait()

    def gather(j, br):
        pltpu.async_copy(h_hbm.at[isb[j]], rows[br], gsem[br])

    def gather_wait(j, br):
        pltpu.make_async_copy(h_hbm.at[isb[j]], rows[br], gsem[br]).wait()

    def scatter(j, br):
        pltpu.async_copy(rows[br], acc.at[idb[j]], ssem[br], add=True)

    def scatter_wait(j, br):
        pltpu.make_async_copy(rows[br], acc.at[idb[j]], ssem[br]).wait()

    idx_load(0, 0)
    idx_load(1, 1)
    idx_load(2, 2)
    idx_wait(0)
    gather(0, 0)

    def outer(t, _):
        for j in range(12):
            g = t * 12 + j
            br = j % 3
            bi = j % 6

            @pl.when(g >= 2)
            def _():
                scatter_wait((j - 2) % 6, (j - 2) % 3)

            @pl.when(g + 3 < nch)
            def _():
                idx_load(g + 3, (j + 3) % 6)

            gather_wait(bi, br)
            scatter(bi, br)

            @pl.when(g + 1 < nch)
            def _():
                idx_wait((j + 1) % 6)
                gather((j + 1) % 6, (j + 1) % 3)
        return 0

    lax.fori_loop(0, nch // 12, outer, 0)
    scatter_wait(4, 1)
    scatter_wait(5, 2)
    plsc.subcore_barrier()

    def wslice(i, _):
        base = s * 640 + i * 80
        pltpu.sync_copy(acc.at[pl.ds(base, 80)], zsrc)
        pltpu.sync_copy(zsrc, out_hbm.at[c].at[pl.ds(base, 80)])
        return 0

    lax.fori_loop(0, 8, wslice, 0)


def _deg_body(dst_hbm, deg_hbm, dacc, buf, idst, sem):
    del sem
    c = lax.axis_index("c")
    s = lax.axis_index("s")
    wid = c * 16 + s

    zero16 = jnp.zeros((16,), jnp.float32)
    one16 = jnp.ones((16,), jnp.float32)

    def fill_zero(i, _):
        for j in range(8):
            buf[i, pl.ds(j * 16, 16)] = zero16
        return 0

    def fill_one(i, _):
        for j in range(8):
            buf[i, pl.ds(j * 16, 16)] = one16
        return 0

    lax.fori_loop(0, 80, fill_zero, 0)
    zsrc = buf.at[pl.ds(0, 80)]

    def zslice(i, _):
        pltpu.sync_copy(zsrc, dacc.at[pl.ds(s * 640 + i * 80, 80)])
        return 0

    lax.fori_loop(0, 8, zslice, 0)
    lax.fori_loop(0, CHUNK, fill_one, 0)
    plsc.subcore_barrier()

    ebase = wid * EDGES_PER_W

    def chunk(g, _):
        off = ebase + g * CHUNK
        pltpu.sync_copy(dst_hbm.at[pl.ds(off, CHUNK)], idst)
        pltpu.sync_copy(buf, dacc.at[idst], add=True)
        return 0

    lax.fori_loop(0, EDGES_PER_W // CHUNK, chunk, 0)
    plsc.subcore_barrier()

    def wslice(i, _):
        base = s * 640 + i * 80
        pltpu.sync_copy(dacc.at[pl.ds(base, 80)], zsrc)
        pltpu.sync_copy(zsrc, deg_hbm.at[c].at[pl.ds(base, 80)])
        return 0

    lax.fori_loop(0, 8, wslice, 0)


@functools.lru_cache(maxsize=None)
def _make_seg_sum():
    mesh = plsc.VectorSubcoreMesh(core_axis_name="c", subcore_axis_name="s")
    return pl.kernel(
        _seg_sum_body,
        out_type=(jax.ShapeDtypeStruct((2, N_PAD, D), jnp.float32),),
        mesh=mesh,
        scratch_types=(
            [pltpu.VMEM_SHARED((N_PAD, D), jnp.float32)]
            + [pltpu.VMEM((CHUNK, D), jnp.float32)] * 3
            + [pltpu.VMEM((CHUNK,), jnp.int32)] * 12
            + [pltpu.SemaphoreType.DMA] * 12
        ),
    )


@functools.lru_cache(maxsize=None)
def _make_deg():
    mesh = plsc.VectorSubcoreMesh(core_axis_name="c", subcore_axis_name="s")
    return pl.kernel(
        _deg_body,
        out_type=(jax.ShapeDtypeStruct((2, N_PAD, D), jnp.float32),),
        mesh=mesh,
        scratch_types=[
            pltpu.VMEM_SHARED((N_PAD, D), jnp.float32),
            pltpu.VMEM((CHUNK, D), jnp.float32),
            pltpu.VMEM((CHUNK,), jnp.int32),
            pltpu.SemaphoreType.DMA,
        ],
    )


def _layer_body(s_ref, h_ref, d_ref, wm_ref, bm_ref, wua_ref, wub_ref, bu_ref, o_ref):
    h = h_ref[...]
    t = s_ref[0] + s_ref[1] + h
    deg = d_ref[0, :, 0:1] + d_ref[1, :, 0:1] + 1.0
    aggr = jax.lax.dot_general(t, wm_ref[...], (((1,), (0,)), ((), ())),
                               preferred_element_type=jnp.float32)
    aggr = aggr + deg * bm_ref[...]
    u = jax.lax.dot_general(aggr, wua_ref[...], (((1,), (0,)), ((), ())),
                            preferred_element_type=jnp.float32)
    u = u + jax.lax.dot_general(h, wub_ref[...], (((1,), (0,)), ((), ())),
                                preferred_element_type=jnp.float32)
    o_ref[...] = jnp.maximum(u + bu_ref[...], 0.0)


_BLK = 256


def _tc_layer(s_parts, h, deg, Wm, bm, Wu, bu):
    grid = (N_PAD // _BLK,)
    return pl.pallas_call(
        _layer_body,
        grid=grid,
        in_specs=[
            pl.BlockSpec((2, _BLK, D), lambda i: (0, i, 0)),
            pl.BlockSpec((_BLK, D), lambda i: (i, 0)),
            pl.BlockSpec((2, _BLK, D), lambda i: (0, i, 0)),
            pl.BlockSpec((D, D), lambda i: (0, 0)),
            pl.BlockSpec((1, D), lambda i: (0, 0)),
            pl.BlockSpec((D, D), lambda i: (0, 0)),
            pl.BlockSpec((D, D), lambda i: (0, 0)),
            pl.BlockSpec((1, D), lambda i: (0, 0)),
        ],
        out_specs=pl.BlockSpec((_BLK, D), lambda i: (i, 0)),
        out_shape=jax.ShapeDtypeStruct((N_PAD, D), jnp.float32),
    )(s_parts, h, deg, Wm, bm.reshape(1, D), Wu[:D], Wu[D:], bu.reshape(1, D))


def kernel(x, edge_index, Wm0, bm0, Wu0, bu0, Wm1, bm1, Wu1, bu1, Wm2, bm2, Wu2, bu2):
    src = edge_index[0].astype(jnp.int32)
    dst = edge_index[1].astype(jnp.int32)
    n_extra = E_PAD - src.shape[0]
    src = jnp.concatenate([src, jnp.zeros((n_extra,), jnp.int32)])
    dst = jnp.concatenate([dst, jnp.full((n_extra,), DUMMY_DST, jnp.int32)])

    h = jnp.concatenate([x, jnp.zeros((N_PAD - N_NODES, D), x.dtype)])

    (deg,) = _make_deg()(dst)
    (s_parts,) = _make_seg_sum()(h, src, dst)
    h = _tc_layer(s_parts, h, deg, Wm0, bm0, Wu0, bu0)
    (s_parts,) = _make_seg_sum()(h, src, dst)
    h = _tc_layer(s_parts, h, deg, Wm1, bm1, Wu1, bu1)
    (s_parts,) = _make_seg_sum()(h, src, dst)
    h = _tc_layer(s_parts, h, deg, Wm2, bm2, Wu2, bu2)
    return h[:N_NODES]
